# baseline (speedup 1.0000x reference)
import functools

import jax
import jax.numpy as jnp
from jax import lax
from jax.experimental import pallas as pl
from jax.experimental.pallas import tpu as pltpu
from jax.experimental.pallas import tpu_sc as plsc

NUM_MODELS = 8
NUM_SPECIES = 4
AEV_DIM = 384
D1, D2, D3 = 160, 128, 96
BLK = 1024
CHUNK = 128
NUM_WORKERS = 32


def _celu(x):
    return jnp.where(x > 0, x, 0.1 * (jnp.exp(x / 0.1) - 1.0))


def _split(x):
    hi = x.astype(jnp.bfloat16)
    lo = (x - hi.astype(jnp.float32)).astype(jnp.bfloat16)
    return hi, lo


def _dot3(xh, xl, wh, wl):
    f32 = jnp.float32
    return (jnp.dot(xh, wh, preferred_element_type=f32)
            + jnp.dot(xl, wh, preferred_element_type=f32)
            + jnp.dot(xh, wl, preferred_element_type=f32))


def _sc_scatter(aev2d, dest1d, n_atoms, capacity, num_chunks):
    max_iters = -(-num_chunks // NUM_WORKERS)
    mesh = plsc.VectorSubcoreMesh(core_axis_name="c", subcore_axis_name="s")

    @functools.partial(
        pl.kernel,
        out_type=jax.ShapeDtypeStruct((capacity, AEV_DIM), jnp.float32),
        mesh=mesh,
        scratch_types=[
            pltpu.VMEM((CHUNK,), jnp.int32),
            pltpu.VMEM((CHUNK, AEV_DIM), jnp.float32),
            pltpu.SemaphoreType.DMA,
        ],
    )
    def scatter_kernel(aev_hbm, dest_hbm, out_hbm, idx_v, rows_v, sem):
        cid = lax.axis_index("c")
        sid = lax.axis_index("s")
        wid = sid * 2 + cid
        for j in range(max_iters):
            i = wid + j * NUM_WORKERS

            @pl.when(i < num_chunks)
            def _():
                start = jnp.minimum(i * CHUNK, n_atoms - CHUNK)
                rows_cp = pltpu.async_copy(
                    aev_hbm.at[pl.ds(start, CHUNK)], rows_v, sem)
                pltpu.sync_copy(dest_hbm.at[pl.ds(start, CHUNK)], idx_v)
                rows_cp.wait()
                pltpu.async_copy(rows_v, out_hbm.at[idx_v], sem).wait()

    return scatter_kernel(aev2d, dest1d)


def _tc_ffn_body(bs_ref, bc_ref, x_ref, w0h_ref, w0l_ref, b0_ref,
                 w1h_ref, w1l_ref, b1_ref, w2h_ref, w2l_ref, b2_ref,
                 w3_ref, b3_ref, out_ref):
    i = pl.program_id(0)

    @pl.when(i == 0)
    def _():
        out_ref[...] = jnp.zeros_like(out_ref)

    count = bc_ref[i]

    @pl.when(count > 0)
    def _():
        x = x_ref[...]
        xh, xl = _split(x)
        h0 = _dot3(xh, xl, w0h_ref[0], w0l_ref[0]) + b0_ref[0]
        h0 = _celu(h0)
        acc = jnp.zeros((BLK, 1), jnp.float32)
        for m in range(NUM_MODELS):
            hm = h0[:, m * D1:(m + 1) * D1]
            hh, hl = _split(hm)
            h = _dot3(hh, hl, w1h_ref[0, m], w1l_ref[0, m])
            h = _celu(h + b1_ref[0, m])
            hh, hl = _split(h)
            h = _dot3(hh, hl, w2h_ref[0, m], w2l_ref[0, m])
            h = _celu(h + b2_ref[0, m])
            w3v = w3_ref[0, m].reshape((1, D3))
            acc = acc + jnp.sum(h * w3v, axis=1, keepdims=True)
        b3_mean = jnp.sum(b3_ref[...]) * (1.0 / NUM_MODELS)
        rowid = lax.broadcasted_iota(jnp.int32, (BLK, 1), 0)
        masked = jnp.where(rowid < count, acc, 0.0)
        blocksum = (jnp.sum(masked) * (1.0 / NUM_MODELS)
                    + count.astype(jnp.float32) * b3_mean)
        out_ref[...] = out_ref[...] + blocksum


def _tc_ffn(gathered, block_species, block_count, w0h, w0l, b0c,
            w1h, w1l, b1r, w2h, w2l, b2r, W3, b3, num_blocks):
    def wspec(shape):
        return pl.BlockSpec((1,) + shape,
                            lambda i, bs, bc: (bs[i],) + (0,) * len(shape))

    grid_spec = pltpu.PrefetchScalarGridSpec(
        num_scalar_prefetch=2,
        grid=(num_blocks,),
        in_specs=[
            pl.BlockSpec((BLK, AEV_DIM), lambda i, bs, bc: (i, 0)),
            wspec((AEV_DIM, NUM_MODELS * D1)),
            wspec((AEV_DIM, NUM_MODELS * D1)),
            wspec((1, NUM_MODELS * D1)),
            wspec((NUM_MODELS, D1, D2)),
            wspec((NUM_MODELS, D1, D2)),
            wspec((NUM_MODELS, 1, D2)),
            wspec((NUM_MODELS, D2, D3)),
            wspec((NUM_MODELS, D2, D3)),
            wspec((NUM_MODELS, 1, D3)),
            wspec((NUM_MODELS, D3, 1)),
            wspec((NUM_MODELS, 1, 1)),
        ],
        out_specs=pl.BlockSpec((1, 1), lambda i, bs, bc: (0, 0)),
    )
    out = pl.pallas_call(
        _tc_ffn_body,
        grid_spec=grid_spec,
        out_shape=jax.ShapeDtypeStruct((1, 1), jnp.float32),
        compiler_params=pltpu.CompilerParams(
            dimension_semantics=("arbitrary",)),
    )(block_species, block_count, gathered,
      w0h, w0l, b0c, w1h, w1l, b1r, w2h, w2l, b2r, W3, b3)
    return out


def kernel(species, aev, W0, b0, W1, b1, W2, b2, W3, b3):
    n = species.shape[-1]
    num_blocks = -(-n // BLK) + NUM_SPECIES - 1
    capacity = num_blocks * BLK
    num_chunks = -(-n // CHUNK)

    sp = species.reshape(-1)
    aev2d = aev.reshape(n, AEV_DIM)

    oh = (sp[:, None] == jnp.arange(NUM_SPECIES)[None, :]).astype(jnp.int32)
    csum = jnp.cumsum(oh, axis=0)
    counts = csum[-1]
    rank = jnp.sum(oh * csum, axis=1) - 1
    nblk = (counts + BLK - 1) // BLK
    blk_bound = jnp.cumsum(nblk)
    pad_start = (blk_bound - nblk) * BLK
    dest = rank + jnp.sum(oh * pad_start[None, :], axis=1)

    bids = jnp.arange(num_blocks, dtype=jnp.int32)
    bs = jnp.minimum(
        jnp.sum((bids[:, None] >= blk_bound[None, :]).astype(jnp.int32),
                axis=1),
        NUM_SPECIES - 1).astype(jnp.int32)
    bstart = jnp.take(blk_bound - nblk, bs)
    bcount = jnp.clip(jnp.take(counts, bs) - (bids - bstart) * BLK,
                      0, BLK).astype(jnp.int32)

    w0c = W0.transpose(0, 2, 1, 3).reshape(NUM_SPECIES, AEV_DIM,
                                           NUM_MODELS * D1)
    b0c = b0.reshape(NUM_SPECIES, 1, NUM_MODELS * D1)
    w0h, w0l = _split(w0c)
    w1h, w1l = _split(W1)
    w2h, w2l = _split(W2)
    b1r = b1.reshape(NUM_SPECIES, NUM_MODELS, 1, D2)
    b2r = b2.reshape(NUM_SPECIES, NUM_MODELS, 1, D3)

    gathered = _sc_scatter(aev2d, dest, n, capacity, num_chunks)
    out = _tc_ffn(gathered, bs, bcount, w0h, w0l, b0c, w1h, w1l, b1r,
                  w2h, w2l, b2r, W3, b3, num_blocks)
    return (species, out.reshape(1))
